# baseline (speedup 1.0000x reference)
import functools

import jax
import jax.numpy as jnp
from jax import lax
from jax.experimental import pallas as pl
from jax.experimental.pallas import tpu as pltpu
from jax.experimental.pallas import tpu_sc as plsc

_DIM = 64
_L = 16
_NW = 32
_BATCH = 16384
_MARGIN = 1.0
_PER_W = _BATCH // _NW
_CH = 128
_NCH = _PER_W // _CH
_GC = _CH // _L


def _rsqrt(x):
    xi = lax.bitcast_convert_type(x, jnp.int32)
    yi = jnp.int32(0x5F3759DF) - (xi >> 1)
    y = lax.bitcast_convert_type(yi, jnp.float32)
    for _ in range(3):
        y = y * (1.5 - 0.5 * x * y * y)
    return y


def _sc_body(ent_hbm, rel_hbm, heads_hbm, rels_hbm, tails_hbm,
             hoff_hbm, roff_hbm, toff_hbm, out_hbm,
             hidx_v, ridx_v, tidx_v, hoff_v, roff_v, toff_v,
             hrows, rrows, trows,
             epos, eneg, ostage, sem):
    wid = lax.axis_index("s") * 2 + lax.axis_index("c")
    iota = lax.iota(jnp.int32, _L)

    for pol in range(2):
        eref = epos if pol == 0 else eneg
        for c in range(_NCH):
            base = pol * _BATCH + wid * _PER_W + c * _CH
            pltpu.sync_copy(heads_hbm.at[pl.ds(base, _CH)], hidx_v)
            pltpu.sync_copy(rels_hbm.at[pl.ds(base, _CH)], ridx_v)
            pltpu.sync_copy(tails_hbm.at[pl.ds(base, _CH)], tidx_v)
            pltpu.sync_copy(hoff_hbm.at[pl.ds(base, _CH)], hoff_v)
            pltpu.sync_copy(roff_hbm.at[pl.ds(base, _CH)], roff_v)
            pltpu.sync_copy(toff_hbm.at[pl.ds(base, _CH)], toff_v)
            cp1 = pltpu.async_copy(ent_hbm.at[hidx_v], hrows, sem)
            cp2 = pltpu.async_copy(rel_hbm.at[ridx_v], rrows, sem)
            cp3 = pltpu.async_copy(ent_hbm.at[tidx_v], trows, sem)
            cp1.wait()
            cp2.wait()
            cp3.wait()

            for g in range(_GC):
                lanes = g * _L + iota
                hof = hoff_v[pl.ds(g * _L, _L)]
                rof = roff_v[pl.ds(g * _L, _L)]
                tof = toff_v[pl.ds(g * _L, _L)]

                def d_body(d, carry):
                    hh, rr, tt, hr, ht, rt = carry
                    hv = plsc.load_gather(hrows, [lanes, hof + d])
                    rv = plsc.load_gather(rrows, [lanes, rof + d])
                    tv = plsc.load_gather(trows, [lanes, tof + d])
                    return (hh + hv * hv, rr + rv * rv, tt + tv * tv,
                            hr + hv * rv, ht + hv * tv, rt + rv * tv)

                z = jnp.zeros((_L,), jnp.float32)
                hh, rr, tt, hr, ht, rt = lax.fori_loop(0, _DIM, d_body,
                                                       (z, z, z, z, z, z))
                ih = _rsqrt(jnp.maximum(hh, 1e-24))
                ir = _rsqrt(jnp.maximum(rr, 1e-24))
                it = _rsqrt(jnp.maximum(tt, 1e-24))
                e2 = (hh * ih * ih + rr * ir * ir + tt * it * it
                      + 2.0 * (hr * (ih * ir) - ht * (ih * it)
                               - rt * (ir * it)))
                e2 = jnp.maximum(e2, 0.0)
                e = e2 * _rsqrt(jnp.maximum(e2, 1e-30))
                eref[pl.ds(c * _CH + g * _L, _L)] = e

    acc = jnp.zeros((_L,), jnp.float32)
    for g in range(_PER_W // _L):
        lp = epos[pl.ds(g * _L, _L)]
        ln = eneg[pl.ds(g * _L, _L)]
        acc = acc + jnp.maximum(_MARGIN + lp - ln, 0.0)
    ostage[...] = acc
    pltpu.sync_copy(ostage, out_hbm.at[wid])


_sc_call = functools.partial(
    pl.kernel,
    mesh=plsc.VectorSubcoreMesh(core_axis_name="c", subcore_axis_name="s"),
    out_type=jax.ShapeDtypeStruct((_NW, _L), jnp.float32),
    scratch_types=[
        pltpu.VMEM((_CH,), jnp.int32),
        pltpu.VMEM((_CH,), jnp.int32),
        pltpu.VMEM((_CH,), jnp.int32),
        pltpu.VMEM((_CH,), jnp.int32),
        pltpu.VMEM((_CH,), jnp.int32),
        pltpu.VMEM((_CH,), jnp.int32),
        pltpu.VMEM((_CH, 2 * _DIM), jnp.float32),
        pltpu.VMEM((_CH, 2 * _DIM), jnp.float32),
        pltpu.VMEM((_CH, 2 * _DIM), jnp.float32),
        pltpu.VMEM((_PER_W,), jnp.float32),
        pltpu.VMEM((_PER_W,), jnp.float32),
        pltpu.VMEM((_L,), jnp.float32),
        pltpu.SemaphoreType.DMA,
    ],
    compiler_params=pltpu.CompilerParams(needs_layout_passes=False,
                                         use_tc_tiling_on_sc=True),
)(_sc_body)


def kernel(pos_triples, neg_triples, ent_emb, rel_emb):
    tri = jnp.concatenate([pos_triples, neg_triples], axis=0).astype(jnp.int32)
    heads = tri[:, 0]
    rels = tri[:, 1]
    tails = tri[:, 2]
    ent2 = ent_emb.reshape(ent_emb.shape[0] // 2, 2 * _DIM)
    rel2 = rel_emb.reshape(rel_emb.shape[0] // 2, 2 * _DIM)
    partials = _sc_call(ent2, rel2,
                        heads >> 1, rels >> 1, tails >> 1,
                        (heads & 1) * _DIM, (rels & 1) * _DIM,
                        (tails & 1) * _DIM)
    return jnp.sum(partials) / jnp.float32(_BATCH)


# speedup vs baseline: 1.8749x; 1.8749x over previous
import functools

import jax
import jax.numpy as jnp
from jax import lax
from jax.experimental import pallas as pl
from jax.experimental.pallas import tpu as pltpu
from jax.experimental.pallas import tpu_sc as plsc

_DIM = 64
_L = 16
_NW = 32
_BATCH = 16384
_MARGIN = 1.0
_PER_W = _BATCH // _NW
_CH = 128
_NCH = _PER_W // _CH
_GC = _CH // _L


_E = 1000000
_K = 501760
_TC = 2048
_TG = _K // _TC


def _relayout_body(e1, e2, r1, r2, eo, ro):
    eo[...] = jnp.concatenate([e1[...].T, e2[...].T], axis=1)
    ro[...] = jnp.concatenate([r1[...].T, r2[...].T], axis=1)


_tc_relayout = pl.pallas_call(
    _relayout_body,
    grid=(_TG,),
    in_specs=[
        pl.BlockSpec((_DIM, _TC), lambda g: (0, g)),
        pl.BlockSpec((_DIM, _TC), lambda g: (0, jnp.minimum(g + _TG, 488))),
        pl.BlockSpec((_DIM, _TC), lambda g: (0, g)),
        pl.BlockSpec((_DIM, _TC), lambda g: (0, jnp.minimum(g + _TG, 488))),
    ],
    out_specs=[
        pl.BlockSpec((_TC, 2 * _DIM), lambda g: (g, 0)),
        pl.BlockSpec((_TC, 2 * _DIM), lambda g: (g, 0)),
    ],
    out_shape=[
        jax.ShapeDtypeStruct((_K, 2 * _DIM), jnp.float32),
        jax.ShapeDtypeStruct((_K, 2 * _DIM), jnp.float32),
    ],
)


def _rsqrt(x):
    xi = lax.bitcast_convert_type(x, jnp.int32)
    yi = jnp.int32(0x5F3759DF) - (xi >> 1)
    y = lax.bitcast_convert_type(yi, jnp.float32)
    for _ in range(3):
        y = y * (1.5 - 0.5 * x * y * y)
    return y


def _sc_body(ent_hbm, rel_hbm, heads_hbm, rels_hbm, tails_hbm,
             hoff_hbm, roff_hbm, toff_hbm, out_hbm,
             hidx_v, ridx_v, tidx_v, hoff_v, roff_v, toff_v,
             hrows, rrows, trows,
             epos, eneg, ostage, sem):
    wid = lax.axis_index("s") * 2 + lax.axis_index("c")
    iota = lax.iota(jnp.int32, _L)

    for pol in range(2):
        eref = epos if pol == 0 else eneg
        for c in range(_NCH):
            base = pol * _BATCH + wid * _PER_W + c * _CH
            pltpu.sync_copy(heads_hbm.at[pl.ds(base, _CH)], hidx_v)
            pltpu.sync_copy(rels_hbm.at[pl.ds(base, _CH)], ridx_v)
            pltpu.sync_copy(tails_hbm.at[pl.ds(base, _CH)], tidx_v)
            pltpu.sync_copy(hoff_hbm.at[pl.ds(base, _CH)], hoff_v)
            pltpu.sync_copy(roff_hbm.at[pl.ds(base, _CH)], roff_v)
            pltpu.sync_copy(toff_hbm.at[pl.ds(base, _CH)], toff_v)
            cp1 = pltpu.async_copy(ent_hbm.at[hidx_v], hrows, sem)
            cp2 = pltpu.async_copy(rel_hbm.at[ridx_v], rrows, sem)
            cp3 = pltpu.async_copy(ent_hbm.at[tidx_v], trows, sem)
            cp1.wait()
            cp2.wait()
            cp3.wait()

            for g in range(_GC):
                lanes = g * _L + iota
                hof = hoff_v[pl.ds(g * _L, _L)]
                rof = roff_v[pl.ds(g * _L, _L)]
                tof = toff_v[pl.ds(g * _L, _L)]

                def d_body(d, carry):
                    hh, rr, tt, hr, ht, rt = carry
                    hv = plsc.load_gather(hrows, [lanes, hof + d])
                    rv = plsc.load_gather(rrows, [lanes, rof + d])
                    tv = plsc.load_gather(trows, [lanes, tof + d])
                    return (hh + hv * hv, rr + rv * rv, tt + tv * tv,
                            hr + hv * rv, ht + hv * tv, rt + rv * tv)

                z = jnp.zeros((_L,), jnp.float32)
                hh, rr, tt, hr, ht, rt = lax.fori_loop(0, _DIM, d_body,
                                                       (z, z, z, z, z, z))
                ih = _rsqrt(jnp.maximum(hh, 1e-24))
                ir = _rsqrt(jnp.maximum(rr, 1e-24))
                it = _rsqrt(jnp.maximum(tt, 1e-24))
                e2 = (hh * ih * ih + rr * ir * ir + tt * it * it
                      + 2.0 * (hr * (ih * ir) - ht * (ih * it)
                               - rt * (ir * it)))
                e2 = jnp.maximum(e2, 0.0)
                e = e2 * _rsqrt(jnp.maximum(e2, 1e-30))
                eref[pl.ds(c * _CH + g * _L, _L)] = e

    acc = jnp.zeros((_L,), jnp.float32)
    for g in range(_PER_W // _L):
        lp = epos[pl.ds(g * _L, _L)]
        ln = eneg[pl.ds(g * _L, _L)]
        acc = acc + jnp.maximum(_MARGIN + lp - ln, 0.0)
    ostage[...] = acc
    pltpu.sync_copy(ostage, out_hbm.at[wid])


_sc_call = functools.partial(
    pl.kernel,
    mesh=plsc.VectorSubcoreMesh(core_axis_name="c", subcore_axis_name="s"),
    out_type=jax.ShapeDtypeStruct((_NW, _L), jnp.float32),
    scratch_types=[
        pltpu.VMEM((_CH,), jnp.int32),
        pltpu.VMEM((_CH,), jnp.int32),
        pltpu.VMEM((_CH,), jnp.int32),
        pltpu.VMEM((_CH,), jnp.int32),
        pltpu.VMEM((_CH,), jnp.int32),
        pltpu.VMEM((_CH,), jnp.int32),
        pltpu.VMEM((_CH, 2 * _DIM), jnp.float32),
        pltpu.VMEM((_CH, 2 * _DIM), jnp.float32),
        pltpu.VMEM((_CH, 2 * _DIM), jnp.float32),
        pltpu.VMEM((_PER_W,), jnp.float32),
        pltpu.VMEM((_PER_W,), jnp.float32),
        pltpu.VMEM((_L,), jnp.float32),
        pltpu.SemaphoreType.DMA,
    ],
    compiler_params=pltpu.CompilerParams(needs_layout_passes=False,
                                         use_tc_tiling_on_sc=True),
)(_sc_body)


def kernel(pos_triples, neg_triples, ent_emb, rel_emb):
    tri = jnp.concatenate([pos_triples, neg_triples], axis=0).astype(jnp.int32)
    heads = tri[:, 0]
    rels = tri[:, 1]
    tails = tri[:, 2]
    ent2, rel2 = _tc_relayout(ent_emb.T, ent_emb.T, rel_emb.T, rel_emb.T)
    hge = (heads >= _K).astype(jnp.int32)
    rge = (rels >= _K).astype(jnp.int32)
    tge = (tails >= _K).astype(jnp.int32)
    partials = _sc_call(ent2, rel2,
                        heads - _K * hge,
                        rels - _K * rge,
                        tails - _K * tge,
                        hge * _DIM, rge * _DIM, tge * _DIM)
    return jnp.sum(partials) / jnp.float32(_BATCH)


# speedup vs baseline: 1.9742x; 1.0530x over previous
import functools

import jax
import jax.numpy as jnp
from jax import lax
from jax.experimental import pallas as pl
from jax.experimental.pallas import tpu as pltpu
from jax.experimental.pallas import tpu_sc as plsc

_DIM = 64
_L = 16
_NW = 32
_BATCH = 16384
_MARGIN = 1.0
_PER_W = _BATCH // _NW
_CH = 128
_NCH = _PER_W // _CH
_GC = _CH // _L


_E = 1000000
_K = 501760
_TC = 2048
_TG = _K // _TC


def _relayout_body(e1, e2, r1, r2, eo, ro):
    eo[...] = jnp.concatenate([e1[...].T, e2[...].T], axis=1)
    ro[...] = jnp.concatenate([r1[...].T, r2[...].T], axis=1)


_tc_relayout = pl.pallas_call(
    _relayout_body,
    grid=(_TG,),
    in_specs=[
        pl.BlockSpec((_DIM, _TC), lambda g: (0, g)),
        pl.BlockSpec((_DIM, _TC), lambda g: (0, jnp.minimum(g + _TG, 488))),
        pl.BlockSpec((_DIM, _TC), lambda g: (0, g)),
        pl.BlockSpec((_DIM, _TC), lambda g: (0, jnp.minimum(g + _TG, 488))),
    ],
    out_specs=[
        pl.BlockSpec((_TC, 2 * _DIM), lambda g: (g, 0)),
        pl.BlockSpec((_TC, 2 * _DIM), lambda g: (g, 0)),
    ],
    out_shape=[
        jax.ShapeDtypeStruct((_K, 2 * _DIM), jnp.float32),
        jax.ShapeDtypeStruct((_K, 2 * _DIM), jnp.float32),
    ],
)


def _rsqrt(x):
    xi = lax.bitcast_convert_type(x, jnp.int32)
    yi = jnp.int32(0x5F3759DF) - (xi >> 1)
    y = lax.bitcast_convert_type(yi, jnp.float32)
    for _ in range(3):
        y = y * (1.5 - 0.5 * x * y * y)
    return y


def _sc_body(ent_hbm, rel_hbm, heads_hbm, rels_hbm, tails_hbm,
             hoff_hbm, roff_hbm, toff_hbm, out_hbm,
             hidx_v, ridx_v, tidx_v, hoff_v, roff_v, toff_v,
             hrows0, rrows0, trows0, hrows1, rrows1, trows1,
             epos, eneg, ostage, isem, sem0, sem1):
    wid = lax.axis_index("s") * 2 + lax.axis_index("c")
    iota = lax.iota(jnp.int32, _L)
    bufs = ((hrows0, rrows0, trows0, sem0), (hrows1, rrows1, trows1, sem1))

    for pol in range(2):
        eref = epos if pol == 0 else eneg
        base = pol * _BATCH + wid * _PER_W
        idx_cps = (
            pltpu.async_copy(heads_hbm.at[pl.ds(base, _PER_W)], hidx_v, isem),
            pltpu.async_copy(rels_hbm.at[pl.ds(base, _PER_W)], ridx_v, isem),
            pltpu.async_copy(tails_hbm.at[pl.ds(base, _PER_W)], tidx_v, isem),
            pltpu.async_copy(hoff_hbm.at[pl.ds(base, _PER_W)], hoff_v, isem),
            pltpu.async_copy(roff_hbm.at[pl.ds(base, _PER_W)], roff_v, isem),
            pltpu.async_copy(toff_hbm.at[pl.ds(base, _PER_W)], toff_v, isem),
        )
        for cp in idx_cps:
            cp.wait()

        def issue(c):
            hb, rb, tb, sm = bufs[c % 2]
            s = c * _CH
            return (
                pltpu.async_copy(ent_hbm.at[hidx_v.at[pl.ds(s, _CH)]], hb, sm),
                pltpu.async_copy(rel_hbm.at[ridx_v.at[pl.ds(s, _CH)]], rb, sm),
                pltpu.async_copy(ent_hbm.at[tidx_v.at[pl.ds(s, _CH)]], tb, sm),
            )

        cps = issue(0)
        for c in range(_NCH):
            nxt = issue(c + 1) if c + 1 < _NCH else None
            for cp in cps:
                cp.wait()
            hb, rb, tb, _ = bufs[c % 2]

            for g in range(_GC):
                lanes = g * _L + iota
                hof = hoff_v[pl.ds(c * _CH + g * _L, _L)]
                rof = roff_v[pl.ds(c * _CH + g * _L, _L)]
                tof = toff_v[pl.ds(c * _CH + g * _L, _L)]

                def d_body(d, carry):
                    hh, rr, tt, hr, ht, rt = carry
                    hv = plsc.load_gather(hb, [lanes, hof + d])
                    rv = plsc.load_gather(rb, [lanes, rof + d])
                    tv = plsc.load_gather(tb, [lanes, tof + d])
                    return (hh + hv * hv, rr + rv * rv, tt + tv * tv,
                            hr + hv * rv, ht + hv * tv, rt + rv * tv)

                z = jnp.zeros((_L,), jnp.float32)
                hh, rr, tt, hr, ht, rt = lax.fori_loop(0, _DIM, d_body,
                                                       (z, z, z, z, z, z))
                ih = _rsqrt(jnp.maximum(hh, 1e-24))
                ir = _rsqrt(jnp.maximum(rr, 1e-24))
                it = _rsqrt(jnp.maximum(tt, 1e-24))
                e2 = (hh * ih * ih + rr * ir * ir + tt * it * it
                      + 2.0 * (hr * (ih * ir) - ht * (ih * it)
                               - rt * (ir * it)))
                e2 = jnp.maximum(e2, 0.0)
                e = e2 * _rsqrt(jnp.maximum(e2, 1e-30))
                eref[pl.ds(c * _CH + g * _L, _L)] = e
            cps = nxt

    acc = jnp.zeros((_L,), jnp.float32)
    for g in range(_PER_W // _L):
        lp = epos[pl.ds(g * _L, _L)]
        ln = eneg[pl.ds(g * _L, _L)]
        acc = acc + jnp.maximum(_MARGIN + lp - ln, 0.0)
    ostage[...] = acc
    pltpu.sync_copy(ostage, out_hbm.at[wid])


_sc_call = functools.partial(
    pl.kernel,
    mesh=plsc.VectorSubcoreMesh(core_axis_name="c", subcore_axis_name="s"),
    out_type=jax.ShapeDtypeStruct((_NW, _L), jnp.float32),
    scratch_types=[
        pltpu.VMEM((_PER_W,), jnp.int32),
        pltpu.VMEM((_PER_W,), jnp.int32),
        pltpu.VMEM((_PER_W,), jnp.int32),
        pltpu.VMEM((_PER_W,), jnp.int32),
        pltpu.VMEM((_PER_W,), jnp.int32),
        pltpu.VMEM((_PER_W,), jnp.int32),
        pltpu.VMEM((_CH, 2 * _DIM), jnp.float32),
        pltpu.VMEM((_CH, 2 * _DIM), jnp.float32),
        pltpu.VMEM((_CH, 2 * _DIM), jnp.float32),
        pltpu.VMEM((_CH, 2 * _DIM), jnp.float32),
        pltpu.VMEM((_CH, 2 * _DIM), jnp.float32),
        pltpu.VMEM((_CH, 2 * _DIM), jnp.float32),
        pltpu.VMEM((_PER_W,), jnp.float32),
        pltpu.VMEM((_PER_W,), jnp.float32),
        pltpu.VMEM((_L,), jnp.float32),
        pltpu.SemaphoreType.DMA,
        pltpu.SemaphoreType.DMA,
        pltpu.SemaphoreType.DMA,
    ],
    compiler_params=pltpu.CompilerParams(needs_layout_passes=False,
                                         use_tc_tiling_on_sc=True),
)(_sc_body)


def kernel(pos_triples, neg_triples, ent_emb, rel_emb):
    tri = jnp.concatenate([pos_triples, neg_triples], axis=0).astype(jnp.int32)
    heads = tri[:, 0]
    rels = tri[:, 1]
    tails = tri[:, 2]
    ent2, rel2 = _tc_relayout(ent_emb.T, ent_emb.T, rel_emb.T, rel_emb.T)
    hge = (heads >= _K).astype(jnp.int32)
    rge = (rels >= _K).astype(jnp.int32)
    tge = (tails >= _K).astype(jnp.int32)
    partials = _sc_call(ent2, rel2,
                        heads - _K * hge,
                        rels - _K * rge,
                        tails - _K * tge,
                        hge * _DIM, rge * _DIM, tge * _DIM)
    return jnp.sum(partials) / jnp.float32(_BATCH)


# speedup vs baseline: 2.2429x; 1.1361x over previous
import functools

import jax
import jax.numpy as jnp
from jax import lax
from jax.experimental import pallas as pl
from jax.experimental.pallas import tpu as pltpu
from jax.experimental.pallas import tpu_sc as plsc

_DIM = 64
_L = 16
_NW = 32
_BATCH = 16384
_MARGIN = 1.0
_PER_W = _BATCH // _NW
_CH = 128
_NCH = _PER_W // _CH
_GC = _CH // _L


_E = 1000000
_K = 503808
_TC = 4096
_TG = _K // _TC
_EDGE = (_E + _TC - 1) // _TC - 1


def _relayout_body(e1, e2, r1, r2, eo, ro):
    eo[...] = jnp.concatenate([e1[...].T, e2[...].T], axis=1)
    ro[...] = jnp.concatenate([r1[...].T, r2[...].T], axis=1)


_tc_relayout = pl.pallas_call(
    _relayout_body,
    grid=(_TG,),
    in_specs=[
        pl.BlockSpec((_DIM, _TC), lambda g: (0, g)),
        pl.BlockSpec((_DIM, _TC), lambda g: (0, jnp.minimum(g + _TG, _EDGE))),
        pl.BlockSpec((_DIM, _TC), lambda g: (0, g)),
        pl.BlockSpec((_DIM, _TC), lambda g: (0, jnp.minimum(g + _TG, _EDGE))),
    ],
    out_specs=[
        pl.BlockSpec((_TC, 2 * _DIM), lambda g: (g, 0)),
        pl.BlockSpec((_TC, 2 * _DIM), lambda g: (g, 0)),
    ],
    out_shape=[
        jax.ShapeDtypeStruct((_K, 2 * _DIM), jnp.float32),
        jax.ShapeDtypeStruct((_K, 2 * _DIM), jnp.float32),
    ],
)


def _rsqrt(x):
    xi = lax.bitcast_convert_type(x, jnp.int32)
    yi = jnp.int32(0x5F3759DF) - (xi >> 1)
    y = lax.bitcast_convert_type(yi, jnp.float32)
    for _ in range(3):
        y = y * (1.5 - 0.5 * x * y * y)
    return y


def _sc_body(ent_hbm, rel_hbm, heads_hbm, rels_hbm, tails_hbm,
             hoff_hbm, roff_hbm, toff_hbm, out_hbm,
             hidx_v, ridx_v, tidx_v, hoff_v, roff_v, toff_v,
             hrows0, rrows0, trows0, hrows1, rrows1, trows1,
             epos, eneg, ostage, isem, sem0, sem1):
    wid = lax.axis_index("s") * 2 + lax.axis_index("c")
    iota = lax.iota(jnp.int32, _L)
    bufs = ((hrows0, rrows0, trows0, sem0), (hrows1, rrows1, trows1, sem1))

    for pol in range(2):
        eref = epos if pol == 0 else eneg
        base = pol * _BATCH + wid * _PER_W
        idx_cps = (
            pltpu.async_copy(heads_hbm.at[pl.ds(base, _PER_W)], hidx_v, isem),
            pltpu.async_copy(rels_hbm.at[pl.ds(base, _PER_W)], ridx_v, isem),
            pltpu.async_copy(tails_hbm.at[pl.ds(base, _PER_W)], tidx_v, isem),
            pltpu.async_copy(hoff_hbm.at[pl.ds(base, _PER_W)], hoff_v, isem),
            pltpu.async_copy(roff_hbm.at[pl.ds(base, _PER_W)], roff_v, isem),
            pltpu.async_copy(toff_hbm.at[pl.ds(base, _PER_W)], toff_v, isem),
        )
        for cp in idx_cps:
            cp.wait()

        def issue(c):
            hb, rb, tb, sm = bufs[c % 2]
            s = c * _CH
            return (
                pltpu.async_copy(ent_hbm.at[hidx_v.at[pl.ds(s, _CH)]], hb, sm),
                pltpu.async_copy(rel_hbm.at[ridx_v.at[pl.ds(s, _CH)]], rb, sm),
                pltpu.async_copy(ent_hbm.at[tidx_v.at[pl.ds(s, _CH)]], tb, sm),
            )

        cps = issue(0)
        for c in range(_NCH):
            nxt = issue(c + 1) if c + 1 < _NCH else None
            for cp in cps:
                cp.wait()
            hb, rb, tb, _ = bufs[c % 2]

            for g in range(_GC):
                lanes = g * _L + iota
                hof = hoff_v[pl.ds(c * _CH + g * _L, _L)]
                rof = roff_v[pl.ds(c * _CH + g * _L, _L)]
                tof = toff_v[pl.ds(c * _CH + g * _L, _L)]

                def d_body(d, carry):
                    hh, rr, tt, hr, ht, rt = carry
                    hv = plsc.load_gather(hb, [lanes, hof + d])
                    rv = plsc.load_gather(rb, [lanes, rof + d])
                    tv = plsc.load_gather(tb, [lanes, tof + d])
                    return (hh + hv * hv, rr + rv * rv, tt + tv * tv,
                            hr + hv * rv, ht + hv * tv, rt + rv * tv)

                z = jnp.zeros((_L,), jnp.float32)
                hh, rr, tt, hr, ht, rt = lax.fori_loop(0, _DIM, d_body,
                                                       (z, z, z, z, z, z))
                ih = _rsqrt(jnp.maximum(hh, 1e-24))
                ir = _rsqrt(jnp.maximum(rr, 1e-24))
                it = _rsqrt(jnp.maximum(tt, 1e-24))
                e2 = (hh * ih * ih + rr * ir * ir + tt * it * it
                      + 2.0 * (hr * (ih * ir) - ht * (ih * it)
                               - rt * (ir * it)))
                e2 = jnp.maximum(e2, 0.0)
                e = e2 * _rsqrt(jnp.maximum(e2, 1e-30))
                eref[pl.ds(c * _CH + g * _L, _L)] = e
            cps = nxt

    acc = jnp.zeros((_L,), jnp.float32)
    for g in range(_PER_W // _L):
        lp = epos[pl.ds(g * _L, _L)]
        ln = eneg[pl.ds(g * _L, _L)]
        acc = acc + jnp.maximum(_MARGIN + lp - ln, 0.0)
    ostage[...] = acc
    pltpu.sync_copy(ostage, out_hbm.at[wid])


_sc_call = functools.partial(
    pl.kernel,
    mesh=plsc.VectorSubcoreMesh(core_axis_name="c", subcore_axis_name="s"),
    out_type=jax.ShapeDtypeStruct((_NW, _L), jnp.float32),
    scratch_types=[
        pltpu.VMEM((_PER_W,), jnp.int32),
        pltpu.VMEM((_PER_W,), jnp.int32),
        pltpu.VMEM((_PER_W,), jnp.int32),
        pltpu.VMEM((_PER_W,), jnp.int32),
        pltpu.VMEM((_PER_W,), jnp.int32),
        pltpu.VMEM((_PER_W,), jnp.int32),
        pltpu.VMEM((_CH, 2 * _DIM), jnp.float32),
        pltpu.VMEM((_CH, 2 * _DIM), jnp.float32),
        pltpu.VMEM((_CH, 2 * _DIM), jnp.float32),
        pltpu.VMEM((_CH, 2 * _DIM), jnp.float32),
        pltpu.VMEM((_CH, 2 * _DIM), jnp.float32),
        pltpu.VMEM((_CH, 2 * _DIM), jnp.float32),
        pltpu.VMEM((_PER_W,), jnp.float32),
        pltpu.VMEM((_PER_W,), jnp.float32),
        pltpu.VMEM((_L,), jnp.float32),
        pltpu.SemaphoreType.DMA,
        pltpu.SemaphoreType.DMA,
        pltpu.SemaphoreType.DMA,
    ],
    compiler_params=pltpu.CompilerParams(needs_layout_passes=False,
                                         use_tc_tiling_on_sc=True),
)(_sc_body)


def kernel(pos_triples, neg_triples, ent_emb, rel_emb):
    tri = jnp.concatenate([pos_triples, neg_triples], axis=0).astype(jnp.int32)
    heads = tri[:, 0]
    rels = tri[:, 1]
    tails = tri[:, 2]
    ent2, rel2 = _tc_relayout(ent_emb.T, ent_emb.T, rel_emb.T, rel_emb.T)
    hge = (heads >= _K).astype(jnp.int32)
    rge = (rels >= _K).astype(jnp.int32)
    tge = (tails >= _K).astype(jnp.int32)
    partials = _sc_call(ent2, rel2,
                        heads - _K * hge,
                        rels - _K * rge,
                        tails - _K * tge,
                        hge * _DIM, rge * _DIM, tge * _DIM)
    return jnp.sum(partials) / jnp.float32(_BATCH)
